# initial kernel scaffold (unmeasured)
import jax
import jax.numpy as jnp
from jax import lax
from jax.experimental import pallas as pl
from jax.experimental.pallas import tpu as pltpu


def kernel(
    x,
):
    def body(*refs):
        pass

    out_shape = jax.ShapeDtypeStruct(..., jnp.float32)
    return pl.pallas_call(body, out_shape=out_shape)(...)


# baseline (device time: 390141 ns/iter reference)
import jax
import jax.numpy as jnp
from jax import lax
from jax.experimental import pallas as pl
from jax.experimental.pallas import tpu as pltpu

M = 8192
N_HALF = 1024
K = 8
CH = M // K


def kernel(x):
    x = x.reshape(M, 2 * N_HALF)

    def body(x_hbm, out_hbm, recv_buf, local_buf, out_stage,
             send_sems, recv_sems, local_sems, out_sems):
        my_x = lax.axis_index("x")
        my_y = lax.axis_index("y")
        my_z = lax.axis_index("z")
        partner = (1 - my_x, my_y, my_z)

        barrier = pltpu.get_barrier_semaphore()
        pl.semaphore_signal(
            barrier, inc=1, device_id=partner,
            device_id_type=pl.DeviceIdType.MESH,
        )
        pl.semaphore_wait(barrier, 1)

        rdmas = []
        for k in range(K):
            rdma = pltpu.make_async_remote_copy(
                src_ref=x_hbm.at[pl.ds(k * CH, CH),
                                 pl.ds((1 - my_x) * N_HALF, N_HALF)],
                dst_ref=recv_buf.at[k],
                send_sem=send_sems.at[k],
                recv_sem=recv_sems.at[k],
                device_id=partner,
                device_id_type=pl.DeviceIdType.MESH,
            )
            rdma.start()
            rdmas.append(rdma)

        def local_copy(k):
            return pltpu.make_async_copy(
                x_hbm.at[pl.ds(k * CH, CH), pl.ds(my_x * N_HALF, N_HALF)],
                local_buf.at[k % 2],
                local_sems.at[k % 2],
            )

        def out_copy(k):
            return pltpu.make_async_copy(
                out_stage.at[k % 2],
                out_hbm.at[pl.ds(k * CH, CH), :],
                out_sems.at[k % 2],
            )

        local_copy(0).start()
        for k in range(K):
            if k + 1 < K:
                local_copy(k + 1).start()
            local_copy(k).wait()
            rdmas[k].wait_recv()
            if k >= 2:
                out_copy(k - 2).wait()
            out_stage[k % 2] = local_buf[k % 2] + recv_buf[k]
            out_copy(k).start()

        out_copy(K - 2).wait()
        out_copy(K - 1).wait()
        for k in range(K):
            rdmas[k].wait_send()

    return pl.pallas_call(
        body,
        out_shape=jax.ShapeDtypeStruct((M, N_HALF), jnp.float32),
        in_specs=[pl.BlockSpec(memory_space=pl.ANY)],
        out_specs=pl.BlockSpec(memory_space=pl.ANY),
        scratch_shapes=[
            pltpu.VMEM((K, CH, N_HALF), jnp.float32),
            pltpu.VMEM((2, CH, N_HALF), jnp.float32),
            pltpu.VMEM((2, CH, N_HALF), jnp.float32),
            pltpu.SemaphoreType.DMA((K,)),
            pltpu.SemaphoreType.DMA((K,)),
            pltpu.SemaphoreType.DMA((2,)),
            pltpu.SemaphoreType.DMA((2,)),
        ],
        compiler_params=pltpu.CompilerParams(
            collective_id=0,
            vmem_limit_bytes=56 * 1024 * 1024,
        ),
    )(x)


# device time: 258735 ns/iter; 1.5079x vs baseline; 1.5079x over previous
import jax
import jax.numpy as jnp
import numpy as np
from jax import lax
from jax.experimental import pallas as pl
from jax.experimental.pallas import tpu as pltpu

M = 8192
N_HALF = 1024
P = 16
SR = M // P
HR = SR // 2
NHOP = P - 1

RING = [(0, 0), (0, 1), (0, 2), (0, 3),
        (1, 3), (1, 2), (1, 1), (2, 1), (2, 2), (2, 3),
        (3, 3), (3, 2), (3, 1), (3, 0), (2, 0), (1, 0)]
STRIPE_OF_POS = np.array([4 * y + z for (y, z) in RING], np.int32)
POS_OF_STRIPE = np.zeros(P, np.int32)
for _i, (_y, _z) in enumerate(RING):
    POS_OF_STRIPE[4 * _y + _z] = _i
RING_Y = np.array([y for (y, _) in RING], np.int32)
RING_Z = np.array([z for (_, z) in RING], np.int32)


def kernel(x):
    x = x.reshape(M, 2 * N_HALF)

    y = lax.axis_index("y")
    z = lax.axis_index("z")
    r = jnp.take(jnp.asarray(POS_OF_STRIPE), 4 * y + z)
    h_ar = jnp.arange(NHOP, dtype=jnp.int32)
    sop = jnp.asarray(STRIPE_OF_POS)
    cw_slots = jnp.take(sop, (r - h_ar) % P)
    ccw_slots = jnp.take(sop, (r + h_ar) % P)
    cw_rslots = jnp.take(sop, (r - 1 - h_ar) % P)
    ccw_rslots = jnp.take(sop, (r + 1 + h_ar) % P)
    params = jnp.stack([
        4 * y + z,
        jnp.take(jnp.asarray(RING_Y), (r + 1) % P),
        jnp.take(jnp.asarray(RING_Z), (r + 1) % P),
        jnp.take(jnp.asarray(RING_Y), (r - 1) % P),
        jnp.take(jnp.asarray(RING_Z), (r - 1) % P),
    ]).astype(jnp.int32)

    def body(x_hbm, params_ref, cw_s, ccw_s, cw_r, ccw_r, out_hbm,
             plane, cross_recv, local_buf,
             cross_sems, local_sem, store_sem,
             cw_send, cw_recv, ccw_send, ccw_recv,
             out_cw_sems, out_ccw_sems, out_own_sem):
        my_x = lax.axis_index("x")
        my_y = lax.axis_index("y")
        my_z = lax.axis_index("z")
        partner = (1 - my_x, my_y, my_z)
        my_stripe = params_ref[0]
        nxt = (my_x, params_ref[1], params_ref[2])
        prv = (my_x, params_ref[3], params_ref[4])
        row0 = my_stripe * SR

        barrier = pltpu.get_barrier_semaphore()
        for dev in (partner, nxt, prv):
            pl.semaphore_signal(
                barrier, inc=1, device_id=dev,
                device_id_type=pl.DeviceIdType.MESH,
            )
        pl.semaphore_wait(barrier, 3)

        cross = pltpu.make_async_remote_copy(
            src_ref=x_hbm.at[pl.ds(row0, SR), pl.ds((1 - my_x) * N_HALF, N_HALF)],
            dst_ref=cross_recv,
            send_sem=cross_sems.at[0],
            recv_sem=cross_sems.at[1],
            device_id=partner,
            device_id_type=pl.DeviceIdType.MESH,
        )
        cross.start()
        lcp = pltpu.make_async_copy(
            x_hbm.at[pl.ds(row0, SR), pl.ds(my_x * N_HALF, N_HALF)],
            local_buf,
            local_sem,
        )
        lcp.start()
        lcp.wait()
        cross.wait_recv()
        local_buf[:, :] = local_buf[:, :] + cross_recv[:, :]
        stp = pltpu.make_async_copy(local_buf, plane.at[my_stripe], store_sem)
        stp.start()
        stp.wait()
        own_out = pltpu.make_async_copy(
            local_buf, out_hbm.at[pl.ds(row0, SR), :], out_own_sem,
        )
        own_out.start()

        rows_cw = pl.ds(0, HR)
        rows_ccw = pl.ds(HR, HR)
        sends = []
        for h in range(NHOP):
            s_cw = pltpu.make_async_remote_copy(
                src_ref=plane.at[cw_s[h], rows_cw, :],
                dst_ref=plane.at[cw_s[h], rows_cw, :],
                send_sem=cw_send.at[h],
                recv_sem=cw_recv.at[h],
                device_id=nxt,
                device_id_type=pl.DeviceIdType.MESH,
            )
            s_cw.start()
            s_ccw = pltpu.make_async_remote_copy(
                src_ref=plane.at[ccw_s[h], rows_ccw, :],
                dst_ref=plane.at[ccw_s[h], rows_ccw, :],
                send_sem=ccw_send.at[h],
                recv_sem=ccw_recv.at[h],
                device_id=prv,
                device_id_type=pl.DeviceIdType.MESH,
            )
            s_ccw.start()
            sends += [s_cw, s_ccw]

            r_cw = pltpu.make_async_remote_copy(
                src_ref=plane.at[cw_r[h], rows_cw, :],
                dst_ref=plane.at[cw_r[h], rows_cw, :],
                send_sem=cw_send.at[h],
                recv_sem=cw_recv.at[h],
                device_id=prv,
                device_id_type=pl.DeviceIdType.MESH,
            )
            r_cw.wait_recv()
            out_cw = pltpu.make_async_copy(
                plane.at[cw_r[h], rows_cw, :],
                out_hbm.at[pl.ds(cw_r[h] * SR, HR), :],
                out_cw_sems.at[h],
            )
            out_cw.start()

            r_ccw = pltpu.make_async_remote_copy(
                src_ref=plane.at[ccw_r[h], rows_ccw, :],
                dst_ref=plane.at[ccw_r[h], rows_ccw, :],
                send_sem=ccw_send.at[h],
                recv_sem=ccw_recv.at[h],
                device_id=nxt,
                device_id_type=pl.DeviceIdType.MESH,
            )
            r_ccw.wait_recv()
            out_ccw = pltpu.make_async_copy(
                plane.at[ccw_r[h], rows_ccw, :],
                out_hbm.at[pl.ds(ccw_r[h] * SR + HR, HR), :],
                out_ccw_sems.at[h],
            )
            out_ccw.start()

        own_out.wait()
        for h in range(NHOP):
            pltpu.make_async_copy(
                plane.at[0, rows_cw, :],
                out_hbm.at[pl.ds(0, HR), :],
                out_cw_sems.at[h],
            ).wait()
            pltpu.make_async_copy(
                plane.at[0, rows_ccw, :],
                out_hbm.at[pl.ds(HR, HR), :],
                out_ccw_sems.at[h],
            ).wait()
        cross.wait_send()
        for s in sends:
            s.wait_send()

    return pl.pallas_call(
        body,
        out_shape=jax.ShapeDtypeStruct((M, N_HALF), jnp.float32),
        in_specs=[
            pl.BlockSpec(memory_space=pl.ANY),
            pl.BlockSpec(memory_space=pltpu.MemorySpace.SMEM),
            pl.BlockSpec(memory_space=pltpu.MemorySpace.SMEM),
            pl.BlockSpec(memory_space=pltpu.MemorySpace.SMEM),
            pl.BlockSpec(memory_space=pltpu.MemorySpace.SMEM),
            pl.BlockSpec(memory_space=pltpu.MemorySpace.SMEM),
        ],
        out_specs=pl.BlockSpec(memory_space=pl.ANY),
        scratch_shapes=[
            pltpu.VMEM((P, SR, N_HALF), jnp.float32),
            pltpu.VMEM((SR, N_HALF), jnp.float32),
            pltpu.VMEM((SR, N_HALF), jnp.float32),
            pltpu.SemaphoreType.DMA((2,)),
            pltpu.SemaphoreType.DMA,
            pltpu.SemaphoreType.DMA,
            pltpu.SemaphoreType.DMA((NHOP,)),
            pltpu.SemaphoreType.DMA((NHOP,)),
            pltpu.SemaphoreType.DMA((NHOP,)),
            pltpu.SemaphoreType.DMA((NHOP,)),
            pltpu.SemaphoreType.DMA((NHOP,)),
            pltpu.SemaphoreType.DMA((NHOP,)),
            pltpu.SemaphoreType.DMA,
        ],
        compiler_params=pltpu.CompilerParams(
            collective_id=0,
            vmem_limit_bytes=56 * 1024 * 1024,
        ),
    )(x, params, cw_slots, ccw_slots, cw_rslots, ccw_rslots)


# device time: 225733 ns/iter; 1.7283x vs baseline; 1.1462x over previous
import jax
import jax.numpy as jnp
import numpy as np
from jax import lax
from jax.experimental import pallas as pl
from jax.experimental.pallas import tpu as pltpu

M = 8192
N_HALF = 1024
P = 16
SR = M // P
HR = SR // 2
NHOP = P - 1
S = 2
CR = HR // S

RING = [(0, 0), (0, 1), (0, 2), (0, 3),
        (1, 3), (1, 2), (1, 1), (2, 1), (2, 2), (2, 3),
        (3, 3), (3, 2), (3, 1), (3, 0), (2, 0), (1, 0)]
STRIPE_OF_POS = np.array([4 * y + z for (y, z) in RING], np.int32)
POS_OF_STRIPE = np.zeros(P, np.int32)
for _i, (_y, _z) in enumerate(RING):
    POS_OF_STRIPE[4 * _y + _z] = _i
RING_Y = np.array([y for (y, _) in RING], np.int32)
RING_Z = np.array([z for (_, z) in RING], np.int32)

def _off(d, s):
    return d * HR + s * CR


def kernel(x):
    x = x.reshape(M, 2 * N_HALF)

    y = lax.axis_index("y")
    z = lax.axis_index("z")
    r = jnp.take(jnp.asarray(POS_OF_STRIPE), 4 * y + z)
    h_ar = jnp.arange(NHOP, dtype=jnp.int32)
    sop = jnp.asarray(STRIPE_OF_POS)
    cw_rslots = jnp.take(sop, (r - 1 - h_ar) % P)
    ccw_rslots = jnp.take(sop, (r + 1 + h_ar) % P)
    params = jnp.stack([
        4 * y + z,
        jnp.take(jnp.asarray(RING_Y), (r + 1) % P),
        jnp.take(jnp.asarray(RING_Z), (r + 1) % P),
        jnp.take(jnp.asarray(RING_Y), (r - 1) % P),
        jnp.take(jnp.asarray(RING_Z), (r - 1) % P),
    ]).astype(jnp.int32)

    def body(x_hbm, params_ref, cw_r, ccw_r, out_hbm,
             plane, cross_recv, local_buf,
             cross_send_sems, cross_recv_sems, local_sems,
             cw_send, cw_recv, ccw_send, ccw_recv,
             out_cw_sems, out_ccw_sems, out_own_sems):
        my_x = lax.axis_index("x")
        my_y = lax.axis_index("y")
        my_z = lax.axis_index("z")
        partner = (1 - my_x, my_y, my_z)
        my_stripe = params_ref[0]
        nxt = (my_x, params_ref[1], params_ref[2])
        prv = (my_x, params_ref[3], params_ref[4])
        row0 = my_stripe * SR

        barrier = pltpu.get_barrier_semaphore()
        for dev in (partner, nxt, prv):
            pl.semaphore_signal(
                barrier, inc=1, device_id=dev,
                device_id_type=pl.DeviceIdType.MESH,
            )
        pl.semaphore_wait(barrier, 3)

        tgt = (nxt, prv)

        crosses = []
        locals_ = []
        for d in range(2):
            for s in range(S):
                i = d * S + s
                off = _off(d, s)
                c = pltpu.make_async_remote_copy(
                    src_ref=x_hbm.at[pl.ds(row0 + off, CR),
                                     pl.ds((1 - my_x) * N_HALF, N_HALF)],
                    dst_ref=cross_recv.at[pl.ds(off, CR), :],
                    send_sem=cross_send_sems.at[i],
                    recv_sem=cross_recv_sems.at[i],
                    device_id=partner,
                    device_id_type=pl.DeviceIdType.MESH,
                )
                c.start()
                lc = pltpu.make_async_copy(
                    x_hbm.at[pl.ds(row0 + off, CR),
                             pl.ds(my_x * N_HALF, N_HALF)],
                    local_buf.at[pl.ds(off, CR), :],
                    local_sems.at[i],
                )
                lc.start()
                crosses.append(c)
                locals_.append(lc)

        sends = []
        sem_send = (cw_send, ccw_send)
        sem_recv = (cw_recv, ccw_recv)
        for d in range(2):
            for s in range(S):
                i = d * S + s
                off = _off(d, s)
                locals_[i].wait()
                crosses[i].wait_recv()
                sub = pl.ds(off, CR)
                local_buf[sub, :] = local_buf[sub, :] + cross_recv[sub, :]
                h0 = pltpu.make_async_remote_copy(
                    src_ref=local_buf.at[sub, :],
                    dst_ref=plane.at[my_stripe, sub, :],
                    send_sem=sem_send[d].at[0, s],
                    recv_sem=sem_recv[d].at[0, s],
                    device_id=tgt[d],
                    device_id_type=pl.DeviceIdType.MESH,
                )
                h0.start()
                sends.append(h0)
                oo = pltpu.make_async_copy(
                    local_buf.at[sub, :],
                    out_hbm.at[pl.ds(row0 + off, CR), :],
                    out_own_sems.at[i],
                )
                oo.start()

        rslot = (cw_r, ccw_r)
        out_sems = (out_cw_sems, out_ccw_sems)
        for h in range(NHOP):
            for s in range(S):
                for d in range(2):
                    slot = rslot[d][h]
                    sub = pl.ds(_off(d, s), CR)
                    rc = pltpu.make_async_remote_copy(
                        src_ref=plane.at[slot, sub, :],
                        dst_ref=plane.at[slot, sub, :],
                        send_sem=sem_send[d].at[h, s],
                        recv_sem=sem_recv[d].at[h, s],
                        device_id=tgt[d],
                        device_id_type=pl.DeviceIdType.MESH,
                    )
                    rc.wait_recv()
                    if h + 1 < NHOP:
                        fw = pltpu.make_async_remote_copy(
                            src_ref=plane.at[slot, sub, :],
                            dst_ref=plane.at[slot, sub, :],
                            send_sem=sem_send[d].at[h + 1, s],
                            recv_sem=sem_recv[d].at[h + 1, s],
                            device_id=tgt[d],
                            device_id_type=pl.DeviceIdType.MESH,
                        )
                        fw.start()
                        sends.append(fw)
                    od = pltpu.make_async_copy(
                        plane.at[slot, sub, :],
                        out_hbm.at[pl.ds(slot * SR + _off(d, s), CR), :],
                        out_sems[d].at[h, s],
                    )
                    od.start()

        for i in range(2 * S):
            pltpu.make_async_copy(
                local_buf.at[pl.ds(0, CR), :],
                out_hbm.at[pl.ds(0, CR), :],
                out_own_sems.at[i],
            ).wait()
        for h in range(NHOP):
            for s in range(S):
                for d in range(2):
                    pltpu.make_async_copy(
                        plane.at[0, pl.ds(0, CR), :],
                        out_hbm.at[pl.ds(0, CR), :],
                        out_sems[d].at[h, s],
                    ).wait()
        for c in crosses:
            c.wait_send()
        for snd in sends:
            snd.wait_send()

    return pl.pallas_call(
        body,
        out_shape=jax.ShapeDtypeStruct((M, N_HALF), jnp.float32),
        in_specs=[
            pl.BlockSpec(memory_space=pl.ANY),
            pl.BlockSpec(memory_space=pltpu.MemorySpace.SMEM),
            pl.BlockSpec(memory_space=pltpu.MemorySpace.SMEM),
            pl.BlockSpec(memory_space=pltpu.MemorySpace.SMEM),
        ],
        out_specs=pl.BlockSpec(memory_space=pl.ANY),
        scratch_shapes=[
            pltpu.VMEM((P, SR, N_HALF), jnp.float32),
            pltpu.VMEM((SR, N_HALF), jnp.float32),
            pltpu.VMEM((SR, N_HALF), jnp.float32),
            pltpu.SemaphoreType.DMA((2 * S,)),
            pltpu.SemaphoreType.DMA((2 * S,)),
            pltpu.SemaphoreType.DMA((2 * S,)),
            pltpu.SemaphoreType.DMA((NHOP, S)),
            pltpu.SemaphoreType.DMA((NHOP, S)),
            pltpu.SemaphoreType.DMA((NHOP, S)),
            pltpu.SemaphoreType.DMA((NHOP, S)),
            pltpu.SemaphoreType.DMA((NHOP, S)),
            pltpu.SemaphoreType.DMA((NHOP, S)),
            pltpu.SemaphoreType.DMA((2 * S,)),
        ],
        compiler_params=pltpu.CompilerParams(
            collective_id=0,
            vmem_limit_bytes=56 * 1024 * 1024,
        ),
    )(x, params, cw_rslots, ccw_rslots)


# device time: 224814 ns/iter; 1.7354x vs baseline; 1.0041x over previous
import jax
import jax.numpy as jnp
import numpy as np
from jax import lax
from jax.experimental import pallas as pl
from jax.experimental.pallas import tpu as pltpu

M = 8192
N_HALF = 1024
P = 16
SR = M // P
HR = SR // 2
NHOP = P - 1
S = 4
CR = HR // S

RING = [(0, 0), (0, 1), (0, 2), (0, 3),
        (1, 3), (1, 2), (1, 1), (2, 1), (2, 2), (2, 3),
        (3, 3), (3, 2), (3, 1), (3, 0), (2, 0), (1, 0)]
STRIPE_OF_POS = np.array([4 * y + z for (y, z) in RING], np.int32)
POS_OF_STRIPE = np.zeros(P, np.int32)
for _i, (_y, _z) in enumerate(RING):
    POS_OF_STRIPE[4 * _y + _z] = _i
RING_Y = np.array([y for (y, _) in RING], np.int32)
RING_Z = np.array([z for (_, z) in RING], np.int32)

def _off(d, s):
    return d * HR + s * CR


def kernel(x):
    x = x.reshape(M, 2 * N_HALF)

    y = lax.axis_index("y")
    z = lax.axis_index("z")
    r = jnp.take(jnp.asarray(POS_OF_STRIPE), 4 * y + z)
    h_ar = jnp.arange(NHOP, dtype=jnp.int32)
    sop = jnp.asarray(STRIPE_OF_POS)
    cw_rslots = jnp.take(sop, (r - 1 - h_ar) % P)
    ccw_rslots = jnp.take(sop, (r + 1 + h_ar) % P)
    params = jnp.stack([
        4 * y + z,
        jnp.take(jnp.asarray(RING_Y), (r + 1) % P),
        jnp.take(jnp.asarray(RING_Z), (r + 1) % P),
        jnp.take(jnp.asarray(RING_Y), (r - 1) % P),
        jnp.take(jnp.asarray(RING_Z), (r - 1) % P),
    ]).astype(jnp.int32)

    def body(x_hbm, params_ref, cw_r, ccw_r, out_hbm,
             plane, cross_recv, local_buf,
             cross_send_sems, cross_recv_sems, local_sems,
             cw_send, cw_recv, ccw_send, ccw_recv,
             out_cw_sems, out_ccw_sems, out_own_sems):
        my_x = lax.axis_index("x")
        my_y = lax.axis_index("y")
        my_z = lax.axis_index("z")
        partner = (1 - my_x, my_y, my_z)
        my_stripe = params_ref[0]
        nxt = (my_x, params_ref[1], params_ref[2])
        prv = (my_x, params_ref[3], params_ref[4])
        row0 = my_stripe * SR

        barrier = pltpu.get_barrier_semaphore()
        for dev in (partner, nxt, prv):
            pl.semaphore_signal(
                barrier, inc=1, device_id=dev,
                device_id_type=pl.DeviceIdType.MESH,
            )
        pl.semaphore_wait(barrier, 3)

        tgt = (nxt, prv)

        crosses = []
        locals_ = []
        for d in range(2):
            for s in range(S):
                i = d * S + s
                off = _off(d, s)
                c = pltpu.make_async_remote_copy(
                    src_ref=x_hbm.at[pl.ds(row0 + off, CR),
                                     pl.ds((1 - my_x) * N_HALF, N_HALF)],
                    dst_ref=cross_recv.at[pl.ds(off, CR), :],
                    send_sem=cross_send_sems.at[i],
                    recv_sem=cross_recv_sems.at[i],
                    device_id=partner,
                    device_id_type=pl.DeviceIdType.MESH,
                )
                c.start()
                lc = pltpu.make_async_copy(
                    x_hbm.at[pl.ds(row0 + off, CR),
                             pl.ds(my_x * N_HALF, N_HALF)],
                    local_buf.at[pl.ds(off, CR), :],
                    local_sems.at[i],
                )
                lc.start()
                crosses.append(c)
                locals_.append(lc)

        sends = []
        sem_send = (cw_send, ccw_send)
        sem_recv = (cw_recv, ccw_recv)
        for d in range(2):
            for s in range(S):
                i = d * S + s
                off = _off(d, s)
                locals_[i].wait()
                crosses[i].wait_recv()
                sub = pl.ds(off, CR)
                local_buf[sub, :] = local_buf[sub, :] + cross_recv[sub, :]
                h0 = pltpu.make_async_remote_copy(
                    src_ref=local_buf.at[sub, :],
                    dst_ref=plane.at[my_stripe, sub, :],
                    send_sem=sem_send[d].at[0, s],
                    recv_sem=sem_recv[d].at[0, s],
                    device_id=tgt[d],
                    device_id_type=pl.DeviceIdType.MESH,
                )
                h0.start()
                sends.append(h0)
                oo = pltpu.make_async_copy(
                    local_buf.at[sub, :],
                    out_hbm.at[pl.ds(row0 + off, CR), :],
                    out_own_sems.at[i],
                )
                oo.start()

        rslot = (cw_r, ccw_r)
        out_sems = (out_cw_sems, out_ccw_sems)
        for h in range(NHOP):
            for s in range(S):
                for d in range(2):
                    slot = rslot[d][h]
                    sub = pl.ds(_off(d, s), CR)
                    rc = pltpu.make_async_remote_copy(
                        src_ref=plane.at[slot, sub, :],
                        dst_ref=plane.at[slot, sub, :],
                        send_sem=sem_send[d].at[h, s],
                        recv_sem=sem_recv[d].at[h, s],
                        device_id=tgt[d],
                        device_id_type=pl.DeviceIdType.MESH,
                    )
                    rc.wait_recv()
                    if h + 1 < NHOP:
                        fw = pltpu.make_async_remote_copy(
                            src_ref=plane.at[slot, sub, :],
                            dst_ref=plane.at[slot, sub, :],
                            send_sem=sem_send[d].at[h + 1, s],
                            recv_sem=sem_recv[d].at[h + 1, s],
                            device_id=tgt[d],
                            device_id_type=pl.DeviceIdType.MESH,
                        )
                        fw.start()
                        sends.append(fw)
                    od = pltpu.make_async_copy(
                        plane.at[slot, sub, :],
                        out_hbm.at[pl.ds(slot * SR + _off(d, s), CR), :],
                        out_sems[d].at[h, s],
                    )
                    od.start()

        for i in range(2 * S):
            pltpu.make_async_copy(
                local_buf.at[pl.ds(0, CR), :],
                out_hbm.at[pl.ds(0, CR), :],
                out_own_sems.at[i],
            ).wait()
        for h in range(NHOP):
            for s in range(S):
                for d in range(2):
                    pltpu.make_async_copy(
                        plane.at[0, pl.ds(0, CR), :],
                        out_hbm.at[pl.ds(0, CR), :],
                        out_sems[d].at[h, s],
                    ).wait()
        for c in crosses:
            c.wait_send()
        for snd in sends:
            snd.wait_send()

    return pl.pallas_call(
        body,
        out_shape=jax.ShapeDtypeStruct((M, N_HALF), jnp.float32),
        in_specs=[
            pl.BlockSpec(memory_space=pl.ANY),
            pl.BlockSpec(memory_space=pltpu.MemorySpace.SMEM),
            pl.BlockSpec(memory_space=pltpu.MemorySpace.SMEM),
            pl.BlockSpec(memory_space=pltpu.MemorySpace.SMEM),
        ],
        out_specs=pl.BlockSpec(memory_space=pl.ANY),
        scratch_shapes=[
            pltpu.VMEM((P, SR, N_HALF), jnp.float32),
            pltpu.VMEM((SR, N_HALF), jnp.float32),
            pltpu.VMEM((SR, N_HALF), jnp.float32),
            pltpu.SemaphoreType.DMA((2 * S,)),
            pltpu.SemaphoreType.DMA((2 * S,)),
            pltpu.SemaphoreType.DMA((2 * S,)),
            pltpu.SemaphoreType.DMA((NHOP, S)),
            pltpu.SemaphoreType.DMA((NHOP, S)),
            pltpu.SemaphoreType.DMA((NHOP, S)),
            pltpu.SemaphoreType.DMA((NHOP, S)),
            pltpu.SemaphoreType.DMA((NHOP, S)),
            pltpu.SemaphoreType.DMA((NHOP, S)),
            pltpu.SemaphoreType.DMA((2 * S,)),
        ],
        compiler_params=pltpu.CompilerParams(
            collective_id=0,
            vmem_limit_bytes=56 * 1024 * 1024,
        ),
    )(x, params, cw_rslots, ccw_rslots)


# device time: 219097 ns/iter; 1.7807x vs baseline; 1.0261x over previous
import jax
import jax.numpy as jnp
from jax import lax
from jax.experimental import pallas as pl
from jax.experimental.pallas import tpu as pltpu

M = 8192
N_HALF = 1024
P = 16
SR = M // P
HR = SR // 2
NHOP = P - 1
S = 2
CR = HR // S

RING = [(0, 0), (0, 1), (0, 2), (0, 3),
        (1, 3), (1, 2), (1, 1), (2, 1), (2, 2), (2, 3),
        (3, 3), (3, 2), (3, 1), (3, 0), (2, 0), (1, 0)]
STRIPE_OF_POS = [4 * y + z for (y, z) in RING]
POS_OF_STRIPE = [0] * P
for _i, _p in enumerate(STRIPE_OF_POS):
    POS_OF_STRIPE[_p] = _i
RING_Y = [y for (y, _) in RING]
RING_Z = [z for (_, z) in RING]


def _lut(table, idx):
    v = jnp.int32(table[0])
    for i in range(1, len(table)):
        v = jnp.where(idx == i, jnp.int32(table[i]), v)
    return v


def _mod16(v):
    return jnp.where(v >= P, v - P, v)


def _off(d, s):
    return d * HR + s * CR


def kernel(x):
    x = x.reshape(M, 2 * N_HALF)

    def body(x_hbm, out_hbm,
             plane, cross_recv, local_buf,
             cross_send_sems, cross_recv_sems, local_sems,
             cw_send, cw_recv, ccw_send, ccw_recv,
             out_cw_sems, out_ccw_sems, out_own_sems):
        my_x = lax.axis_index("x")
        my_y = lax.axis_index("y")
        my_z = lax.axis_index("z")
        partner = (1 - my_x, my_y, my_z)
        my_stripe = 4 * my_y + my_z
        r = _lut(POS_OF_STRIPE, my_stripe)
        rp1 = _mod16(r + 1)
        rm1 = _mod16(r + NHOP)
        nxt = (my_x, _lut(RING_Y, rp1), _lut(RING_Z, rp1))
        prv = (my_x, _lut(RING_Y, rm1), _lut(RING_Z, rm1))
        cw_rslot = [_lut(STRIPE_OF_POS, _mod16(r + NHOP - h)) for h in range(NHOP)]
        ccw_rslot = [_lut(STRIPE_OF_POS, _mod16(_mod16(r + 1 + h))) for h in range(NHOP)]
        row0 = my_stripe * SR

        barrier = pltpu.get_barrier_semaphore()
        for dev in (partner, nxt, prv):
            pl.semaphore_signal(
                barrier, inc=1, device_id=dev,
                device_id_type=pl.DeviceIdType.MESH,
            )
        pl.semaphore_wait(barrier, 3)

        tgt = (nxt, prv)

        crosses = []
        locals_ = []
        for d in range(2):
            for s in range(S):
                i = d * S + s
                off = _off(d, s)
                c = pltpu.make_async_remote_copy(
                    src_ref=x_hbm.at[pl.ds(row0 + off, CR),
                                     pl.ds((1 - my_x) * N_HALF, N_HALF)],
                    dst_ref=cross_recv.at[pl.ds(off, CR), :],
                    send_sem=cross_send_sems.at[i],
                    recv_sem=cross_recv_sems.at[i],
                    device_id=partner,
                    device_id_type=pl.DeviceIdType.MESH,
                )
                c.start()
                lc = pltpu.make_async_copy(
                    x_hbm.at[pl.ds(row0 + off, CR),
                             pl.ds(my_x * N_HALF, N_HALF)],
                    local_buf.at[pl.ds(off, CR), :],
                    local_sems.at[i],
                )
                lc.start()
                crosses.append(c)
                locals_.append(lc)

        sends = []
        sem_send = (cw_send, ccw_send)
        sem_recv = (cw_recv, ccw_recv)
        for d in range(2):
            for s in range(S):
                i = d * S + s
                off = _off(d, s)
                locals_[i].wait()
                crosses[i].wait_recv()
                sub = pl.ds(off, CR)
                local_buf[sub, :] = local_buf[sub, :] + cross_recv[sub, :]
                h0 = pltpu.make_async_remote_copy(
                    src_ref=local_buf.at[sub, :],
                    dst_ref=plane.at[my_stripe, sub, :],
                    send_sem=sem_send[d].at[0, s],
                    recv_sem=sem_recv[d].at[0, s],
                    device_id=tgt[d],
                    device_id_type=pl.DeviceIdType.MESH,
                )
                h0.start()
                sends.append(h0)
                oo = pltpu.make_async_copy(
                    local_buf.at[sub, :],
                    out_hbm.at[pl.ds(row0 + off, CR), :],
                    out_own_sems.at[i],
                )
                oo.start()

        rslot = (cw_rslot, ccw_rslot)
        out_sems = (out_cw_sems, out_ccw_sems)
        for h in range(NHOP):
            for s in range(S):
                for d in range(2):
                    slot = rslot[d][h]
                    sub = pl.ds(_off(d, s), CR)
                    rc = pltpu.make_async_remote_copy(
                        src_ref=plane.at[slot, sub, :],
                        dst_ref=plane.at[slot, sub, :],
                        send_sem=sem_send[d].at[h, s],
                        recv_sem=sem_recv[d].at[h, s],
                        device_id=tgt[d],
                        device_id_type=pl.DeviceIdType.MESH,
                    )
                    rc.wait_recv()
                    if h + 1 < NHOP:
                        fw = pltpu.make_async_remote_copy(
                            src_ref=plane.at[slot, sub, :],
                            dst_ref=plane.at[slot, sub, :],
                            send_sem=sem_send[d].at[h + 1, s],
                            recv_sem=sem_recv[d].at[h + 1, s],
                            device_id=tgt[d],
                            device_id_type=pl.DeviceIdType.MESH,
                        )
                        fw.start()
                        sends.append(fw)
                    od = pltpu.make_async_copy(
                        plane.at[slot, sub, :],
                        out_hbm.at[pl.ds(slot * SR + _off(d, s), CR), :],
                        out_sems[d].at[h, s],
                    )
                    od.start()

        for i in range(2 * S):
            pltpu.make_async_copy(
                local_buf.at[pl.ds(0, CR), :],
                out_hbm.at[pl.ds(0, CR), :],
                out_own_sems.at[i],
            ).wait()
        for h in range(NHOP):
            for s in range(S):
                for d in range(2):
                    pltpu.make_async_copy(
                        plane.at[0, pl.ds(0, CR), :],
                        out_hbm.at[pl.ds(0, CR), :],
                        out_sems[d].at[h, s],
                    ).wait()
        for c in crosses:
            c.wait_send()
        for snd in sends:
            snd.wait_send()

    return pl.pallas_call(
        body,
        out_shape=jax.ShapeDtypeStruct((M, N_HALF), jnp.float32),
        in_specs=[pl.BlockSpec(memory_space=pl.ANY)],
        out_specs=pl.BlockSpec(memory_space=pl.ANY),
        scratch_shapes=[
            pltpu.VMEM((P, SR, N_HALF), jnp.float32),
            pltpu.VMEM((SR, N_HALF), jnp.float32),
            pltpu.VMEM((SR, N_HALF), jnp.float32),
            pltpu.SemaphoreType.DMA((2 * S,)),
            pltpu.SemaphoreType.DMA((2 * S,)),
            pltpu.SemaphoreType.DMA((2 * S,)),
            pltpu.SemaphoreType.DMA((NHOP, S)),
            pltpu.SemaphoreType.DMA((NHOP, S)),
            pltpu.SemaphoreType.DMA((NHOP, S)),
            pltpu.SemaphoreType.DMA((NHOP, S)),
            pltpu.SemaphoreType.DMA((NHOP, S)),
            pltpu.SemaphoreType.DMA((NHOP, S)),
            pltpu.SemaphoreType.DMA((2 * S,)),
        ],
        compiler_params=pltpu.CompilerParams(
            collective_id=0,
            vmem_limit_bytes=56 * 1024 * 1024,
        ),
    )(x)
